# Initial kernel scaffold; baseline (speedup 1.0000x reference)
import jax
import jax.numpy as jnp
from jax.experimental import pallas as pl


def kernel(inputs, W_route, b_route, W1, b1, W2, b2):
    raise NotImplementedError("write your pallas kernel here")


# speedup vs baseline: 2.8524x; 2.8524x over previous
import jax
import jax.numpy as jnp
from jax.experimental import pallas as pl
from jax.experimental.pallas import tpu as pltpu

E = 8
D = 64
T = 8192
C = 1024
CH = 1024
NSTEP = T // CH


def _body(x_ref, wr_ref, br_ref, w1_ref, b1_ref, w2_ref, b2_ref, l_ref,
          o_ref, cnt_ref):
    i = pl.program_id(0)

    @pl.when(i == 0)
    def _init():
        cnt_ref[...] = jnp.zeros_like(cnt_ref)

    x = x_ref[...]
    logits = jnp.dot(x, wr_ref[...],
                     preferred_element_type=jnp.float32) + br_ref[...]
    m = jnp.max(logits, axis=-1, keepdims=True)
    denom = jnp.sum(jnp.exp(logits - m), axis=-1, keepdims=True)
    gate = 1.0 / denom
    lane = jax.lax.broadcasted_iota(jnp.int32, (CH, E), 1)
    idx = jnp.min(jnp.where(logits >= m, lane, E), axis=-1,
                  keepdims=True)
    onehot = (lane == idx).astype(jnp.float32)

    csum = jnp.dot(l_ref[...], onehot, preferred_element_type=jnp.float32)
    pos = jnp.sum((csum + cnt_ref[...]) * onehot, axis=-1, keepdims=True)
    cnt_ref[...] = cnt_ref[...] + jnp.sum(onehot, axis=0, keepdims=True)
    kept = (pos < float(C)).astype(jnp.float32)
    g = gate * kept

    h = jnp.maximum(
        jnp.dot(x * g, w1_ref[...], preferred_element_type=jnp.float32)
        + b1_ref[...], 0.0)
    lane_e = jax.lax.broadcasted_iota(jnp.int32, (CH, E * D), 1) // D
    hm = jnp.where(lane_e == idx, h, 0.0) * g
    o = jnp.dot(hm, w2_ref[...], preferred_element_type=jnp.float32)
    b2sel = jnp.dot(onehot, b2_ref[...], preferred_element_type=jnp.float32)
    o_ref[...] = o + b2sel * g


def kernel(inputs, W_route, b_route, W1, b1, W2, b2):
    x = inputs.reshape(T, D)
    w1c = W1.transpose(1, 0, 2).reshape(D, E * D)
    b1c = b1.reshape(1, E * D)
    w2c = W2.reshape(E * D, D)
    tril = jnp.tril(jnp.ones((CH, CH), jnp.float32))

    out = pl.pallas_call(
        _body,
        grid=(NSTEP,),
        in_specs=[
            pl.BlockSpec((CH, D), lambda i: (i, 0)),
            pl.BlockSpec((D, E), lambda i: (0, 0)),
            pl.BlockSpec((1, E), lambda i: (0, 0)),
            pl.BlockSpec((D, E * D), lambda i: (0, 0)),
            pl.BlockSpec((1, E * D), lambda i: (0, 0)),
            pl.BlockSpec((E * D, D), lambda i: (0, 0)),
            pl.BlockSpec((E, D), lambda i: (0, 0)),
            pl.BlockSpec((CH, CH), lambda i: (0, 0)),
        ],
        out_specs=pl.BlockSpec((CH, D), lambda i: (i, 0)),
        out_shape=jax.ShapeDtypeStruct((T, D), jnp.float32),
        scratch_shapes=[pltpu.VMEM((1, E), jnp.float32)],
        compiler_params=pltpu.CompilerParams(
            dimension_semantics=("arbitrary",)),
    )(x, W_route, b_route.reshape(1, E), w1c, b1c, w2c, b2, tril)
    return out.reshape(inputs.shape)
